# baseline (speedup 1.0000x reference)
import functools

import jax
import jax.numpy as jnp
from jax import lax
from jax.experimental import pallas as pl
from jax.experimental.pallas import tpu as pltpu
from jax.experimental.pallas import tpu_sc as plsc

_NC = 2
_NS = 16
_NW = _NC * _NS
_CB = 80


def _proj_body(x_ref, a_ref, b_ref, tab_ref):
    x = x_ref[...]
    tab_ref[0] = jnp.dot(x, a_ref[...], preferred_element_type=jnp.float32)
    tab_ref[1] = jnp.dot(x, b_ref[...], preferred_element_type=jnp.float32)


def _tc_proj(x, a, b, bn):
    n, h = x.shape
    grid = (n // bn,)
    row = lambda i: (i, 0)
    zero = lambda i: (0, 0)
    return pl.pallas_call(
        _proj_body,
        grid=grid,
        in_specs=[
            pl.BlockSpec((bn, h), row),
            pl.BlockSpec((h, h), zero),
            pl.BlockSpec((h, h), zero),
        ],
        out_specs=pl.BlockSpec((2, bn, h), lambda i: (0, i, 0)),
        out_shape=jax.ShapeDtypeStruct((2, n, h), jnp.float32),
    )(x, a, b)


def _edge_body(g_ref, e_ref, c_ref, w2_ref, b1_ref, b2_ref, out_ref):
    e = e_ref[...]
    pre = (
        g_ref[0]
        + g_ref[1]
        + jnp.dot(e.astype(jnp.bfloat16), c_ref[...],
                  preferred_element_type=jnp.float32)
        + b1_ref[...]
    )
    h = jnp.maximum(pre, 0.0)
    out_ref[...] = (
        jnp.dot(h.astype(jnp.bfloat16), w2_ref[...],
                preferred_element_type=jnp.float32)
        + b2_ref[...]
        + e
    )


def _tc_edge(g, e, c, w2, b1, b2, be):
    m, h = e.shape
    grid = (m // be,)
    row = lambda i: (i, 0)
    zero = lambda i: (0, 0)
    return pl.pallas_call(
        _edge_body,
        grid=grid,
        in_specs=[
            pl.BlockSpec((2, be, h), lambda i: (0, i, 0)),
            pl.BlockSpec((be, h), row),
            pl.BlockSpec((h, h), zero),
            pl.BlockSpec((h, h), zero),
            pl.BlockSpec((1, h), zero),
            pl.BlockSpec((1, h), zero),
        ],
        out_specs=pl.BlockSpec((be, h), row),
        out_shape=jax.ShapeDtypeStruct((m, h), jnp.float32),
    )(g, e, c, w2, b1, b2)


def _node_body(x_ref, p_ref, d_ref, f_ref, w2_ref, b1_ref, b2_ref, out_ref):
    x = x_ref[...]
    agg = p_ref[0] + p_ref[1]
    pre = (
        jnp.dot(x, d_ref[...], preferred_element_type=jnp.float32)
        + jnp.dot(agg, f_ref[...], preferred_element_type=jnp.float32)
        + b1_ref[...]
    )
    h = jnp.maximum(pre, 0.0)
    out_ref[...] = (
        jnp.dot(h, w2_ref[...], preferred_element_type=jnp.float32)
        + b2_ref[...]
        + x
    )


def _tc_node(x, part, d, f, w2, b1, b2, bn):
    n, h = x.shape
    grid = (n // bn,)
    row = lambda i: (i, 0)
    zero = lambda i: (0, 0)
    return pl.pallas_call(
        _node_body,
        grid=grid,
        in_specs=[
            pl.BlockSpec((bn, h), row),
            pl.BlockSpec((2, bn, h), lambda i: (0, i, 0)),
            pl.BlockSpec((h, h), zero),
            pl.BlockSpec((h, h), zero),
            pl.BlockSpec((h, h), zero),
            pl.BlockSpec((1, h), zero),
            pl.BlockSpec((1, h), zero),
        ],
        out_specs=pl.BlockSpec((bn, h), row),
        out_shape=jax.ShapeDtypeStruct((n, h), jnp.float32),
    )(x, part, d, f, w2, b1, b2)


_NSLOT = 5
_CBS = 40


def _sc_gather(tab, idx4):
    n, h = tab.shape[1], tab.shape[2]
    nch = idx4.shape[2]
    ept = nch * _CB
    e = ept * _NS
    mesh = plsc.VectorSubcoreMesh(core_axis_name="c", subcore_axis_name="s")

    @functools.partial(
        pl.kernel,
        mesh=mesh,
        out_type=jax.ShapeDtypeStruct((2, e, h), jnp.float32),
        scratch_types=[
            pltpu.VMEM((nch, _CB), jnp.int32),
            pltpu.VMEM((_NSLOT, _CB, h), jnp.float32),
            pltpu.SemaphoreType.DMA,
            pltpu.SemaphoreType.DMA((_NSLOT,)),
            pltpu.SemaphoreType.DMA((_NSLOT,)),
        ],
    )
    def gk(tab_hbm, idx_hbm, g_hbm, islab, bufs, sem_i, sem_g, sem_w):
        c = lax.axis_index("c")
        s = lax.axis_index("s")
        table = tab_hbm.at[c]
        cp_idx = pltpu.async_copy(idx_hbm.at[c, s], islab, sem_i)
        cp_idx.wait()

        base0 = s * ept

        def gather_issue(i, slot):
            pltpu.async_copy(table.at[islab.at[i]], bufs.at[slot],
                             sem_g.at[slot])

        def gather_wait(i, slot):
            pltpu.make_async_copy(table.at[islab.at[i]], bufs.at[slot],
                                  sem_g.at[slot]).wait()

        def write_issue(i, slot):
            pltpu.async_copy(bufs.at[slot],
                             g_hbm.at[c, pl.ds(base0 + i * _CB, _CB)],
                             sem_w.at[slot])

        def write_wait(i, slot):
            pltpu.make_async_copy(bufs.at[slot],
                                  g_hbm.at[c, pl.ds(base0 + i * _CB, _CB)],
                                  sem_w.at[slot]).wait()

        gather_issue(0, 0)
        gather_issue(1, 1)

        def body(grp, carry):
            for b in range(_NSLOT):
                i = grp * _NSLOT + b
                gather_wait(i, b)
                write_issue(i, b)
                nb = (b + 2) % _NSLOT

                @pl.when(i >= 3)
                def _():
                    write_wait(i - 3, nb)

                @pl.when(i + 2 < nch)
                def _():
                    gather_issue(i + 2, nb)
            return carry

        lax.fori_loop(0, nch // _NSLOT, body, 0)
        for k in range(3):
            write_wait(nch - 1 - k, (nch - 1 - k) % _NSLOT)

    return gk(tab, idx4)


def _sc_scatter(e_new, idx3, n_nodes):
    e, h = e_new.shape
    nch = idx3.shape[1]
    epw = nch * _CBS
    zr = 32
    n_pad = ((n_nodes + zr * _NS - 1) // (zr * _NS)) * (zr * _NS)
    rpt = n_pad // _NS
    mesh = plsc.VectorSubcoreMesh(core_axis_name="c", subcore_axis_name="s")

    @functools.partial(
        pl.kernel,
        mesh=mesh,
        out_type=jax.ShapeDtypeStruct((_NC, n_pad, h), jnp.float32),
        scratch_types=[
            pltpu.VMEM((nch, _CBS), jnp.int32),
            pltpu.VMEM((2, _CBS, h), jnp.float32),
            pltpu.VMEM((zr, h), jnp.float32),
            pltpu.VMEM_SHARED((n_pad, h), jnp.float32),
            pltpu.SemaphoreType.DMA,
            pltpu.SemaphoreType.DMA((2,)),
            pltpu.SemaphoreType.DMA((2,)),
        ],
    )
    def sk(e_hbm, idx_hbm, out_hbm, islab, bufs, zbuf, agg,
           sem_i, sem_l, sem_s):
        c = lax.axis_index("c")
        s = lax.axis_index("s")
        wid = s * _NC + c
        cp_idx = pltpu.async_copy(idx_hbm.at[wid], islab, sem_i)

        for r in range(zr):
            for k in range(h // 16):
                zbuf[r, pl.ds(k * 16, 16)] = jnp.zeros((16,), jnp.float32)
        for j in range(rpt // zr):
            pltpu.sync_copy(zbuf, agg.at[pl.ds(s * rpt + j * zr, zr)])
        cp_idx.wait()
        plsc.subcore_barrier()

        base0 = wid * epw

        def load_issue(i, slot):
            pltpu.async_copy(e_hbm.at[pl.ds(base0 + i * _CBS, _CBS)],
                             bufs.at[slot], sem_l.at[slot])

        def load_wait(i, slot):
            pltpu.make_async_copy(e_hbm.at[pl.ds(base0 + i * _CBS, _CBS)],
                                  bufs.at[slot], sem_l.at[slot]).wait()

        def scat_issue(i, slot):
            pltpu.async_copy(bufs.at[slot], agg.at[islab.at[i]],
                             sem_s.at[slot], add=True)

        def scat_wait(i, slot):
            pltpu.make_async_copy(bufs.at[slot], agg.at[islab.at[i]],
                                  sem_s.at[slot]).wait()

        load_issue(0, 0)

        def body(grp, carry):
            for b in range(2):
                i = grp * 2 + b
                load_wait(i, b)
                scat_issue(i, b)
                nb = 1 - b

                @pl.when(i >= 1)
                def _():
                    scat_wait(i - 1, nb)

                @pl.when(i + 1 < nch)
                def _():
                    load_issue(i + 1, nb)
            return carry

        lax.fori_loop(0, nch // 2, body, 0)
        scat_wait(nch - 1, (nch - 1) % 2)
        plsc.subcore_barrier()
        pltpu.sync_copy(
            agg.at[pl.ds(s * rpt, rpt)],
            out_hbm.at[c, pl.ds(s * rpt, rpt)],
        )

    return sk(e_new, idx3)


def kernel(x, edge_attr, edge_index, params):
    n, h = x.shape
    e = edge_attr.shape[0]
    src = edge_index[0]
    dst = edge_index[1]
    bn = 2000
    be = 2000

    nch_g = e // (_NS * _CB)
    nch_s = e // (_NW * _CBS)
    idx4 = jnp.stack([src.reshape(_NS, nch_g, _CB),
                      dst.reshape(_NS, nch_g, _CB)])
    idx3 = dst.reshape(_NW, nch_s, _CBS)

    cx, ce = x, edge_attr
    for p in params:
        we1 = p["We1"]
        a, b, c = we1[0:h], we1[h:2 * h], we1[2 * h:3 * h]
        wn1 = p["Wn1"]
        d, f = wn1[0:h], wn1[h:2 * h]
        be1 = p["be1"].reshape(1, h)
        be2 = p["be2"].reshape(1, h)
        bn1 = p["bn1"].reshape(1, h)
        bn2 = p["bn2"].reshape(1, h)

        tab = _tc_proj(cx, a, b, bn)
        g = _sc_gather(tab, idx4)
        ce = _tc_edge(g, ce, c.astype(jnp.bfloat16),
                      p["We2"].astype(jnp.bfloat16), be1, be2, be)
        part = _sc_scatter(ce, idx3, n)
        cx = _tc_node(cx, part, d, f, p["Wn2"], bn1, bn2, bn)

    return (cx, ce)


# speedup vs baseline: 1.0843x; 1.0843x over previous
import functools

import jax
import jax.numpy as jnp
from jax import lax
from jax.experimental import pallas as pl
from jax.experimental.pallas import tpu as pltpu
from jax.experimental.pallas import tpu_sc as plsc

_NC = 2
_NS = 16
_NW = _NC * _NS
_CB = 80


def _proj_body(x_ref, a_ref, b_ref, tab_ref):
    x = x_ref[...]
    tab_ref[0] = jnp.dot(x, a_ref[...], preferred_element_type=jnp.float32)
    tab_ref[1] = jnp.dot(x, b_ref[...], preferred_element_type=jnp.float32)


def _tc_proj(x, a, b, bn):
    n, h = x.shape
    grid = (n // bn,)
    row = lambda i: (i, 0)
    zero = lambda i: (0, 0)
    return pl.pallas_call(
        _proj_body,
        grid=grid,
        in_specs=[
            pl.BlockSpec((bn, h), row),
            pl.BlockSpec((h, h), zero),
            pl.BlockSpec((h, h), zero),
        ],
        out_specs=pl.BlockSpec((2, bn, h), lambda i: (0, i, 0)),
        out_shape=jax.ShapeDtypeStruct((2, n, h), jnp.float32),
    )(x, a, b)


def _edge_body(g_ref, e_ref, c_ref, w2_ref, b1_ref, b2_ref, out_ref):
    e = e_ref[...]
    pre = (
        g_ref[0]
        + g_ref[1]
        + jnp.dot(e, c_ref[...], preferred_element_type=jnp.float32)
        + b1_ref[...]
    )
    h = jnp.maximum(pre, 0.0)
    out_ref[...] = (
        jnp.dot(h, w2_ref[...], preferred_element_type=jnp.float32)
        + b2_ref[...]
        + e
    )


def _tc_edge(g, e, c, w2, b1, b2, be):
    m, h = e.shape
    grid = (m // be,)
    row = lambda i: (i, 0)
    zero = lambda i: (0, 0)
    return pl.pallas_call(
        _edge_body,
        grid=grid,
        in_specs=[
            pl.BlockSpec((2, be, h), lambda i: (0, i, 0)),
            pl.BlockSpec((be, h), row),
            pl.BlockSpec((h, h), zero),
            pl.BlockSpec((h, h), zero),
            pl.BlockSpec((1, h), zero),
            pl.BlockSpec((1, h), zero),
        ],
        out_specs=pl.BlockSpec((be, h), row),
        out_shape=jax.ShapeDtypeStruct((m, h), jnp.float32),
    )(g, e, c, w2, b1, b2)


def _node_body(x_ref, p_ref, d_ref, f_ref, w2_ref, b1_ref, b2_ref, out_ref):
    x = x_ref[...]
    agg = p_ref[0] + p_ref[1]
    pre = (
        jnp.dot(x, d_ref[...], preferred_element_type=jnp.float32)
        + jnp.dot(agg, f_ref[...], preferred_element_type=jnp.float32)
        + b1_ref[...]
    )
    h = jnp.maximum(pre, 0.0)
    out_ref[...] = (
        jnp.dot(h, w2_ref[...], preferred_element_type=jnp.float32)
        + b2_ref[...]
        + x
    )


def _tc_node(x, part, d, f, w2, b1, b2, bn):
    n, h = x.shape
    grid = (n // bn,)
    row = lambda i: (i, 0)
    zero = lambda i: (0, 0)
    return pl.pallas_call(
        _node_body,
        grid=grid,
        in_specs=[
            pl.BlockSpec((bn, h), row),
            pl.BlockSpec((2, bn, h), lambda i: (0, i, 0)),
            pl.BlockSpec((h, h), zero),
            pl.BlockSpec((h, h), zero),
            pl.BlockSpec((h, h), zero),
            pl.BlockSpec((1, h), zero),
            pl.BlockSpec((1, h), zero),
        ],
        out_specs=pl.BlockSpec((bn, h), row),
        out_shape=jax.ShapeDtypeStruct((n, h), jnp.float32),
    )(x, part, d, f, w2, b1, b2)


_NSLOT = 5
_CBS = 40


def _sc_gather(tab, idx4):
    n, h = tab.shape[1], tab.shape[2]
    nch = idx4.shape[2]
    ept = nch * _CB
    e = ept * _NS
    mesh = plsc.VectorSubcoreMesh(core_axis_name="c", subcore_axis_name="s")

    @functools.partial(
        pl.kernel,
        mesh=mesh,
        out_type=jax.ShapeDtypeStruct((2, e, h), jnp.float32),
        scratch_types=[
            pltpu.VMEM((nch, _CB), jnp.int32),
            pltpu.VMEM((_NSLOT, _CB, h), jnp.float32),
            pltpu.SemaphoreType.DMA,
            pltpu.SemaphoreType.DMA((_NSLOT,)),
            pltpu.SemaphoreType.DMA((_NSLOT,)),
        ],
    )
    def gk(tab_hbm, idx_hbm, g_hbm, islab, bufs, sem_i, sem_g, sem_w):
        c = lax.axis_index("c")
        s = lax.axis_index("s")
        table = tab_hbm.at[c]
        cp_idx = pltpu.async_copy(idx_hbm.at[c, s], islab, sem_i)
        cp_idx.wait()

        base0 = s * ept

        def gather_issue(i, slot):
            pltpu.async_copy(table.at[islab.at[i]], bufs.at[slot],
                             sem_g.at[slot])

        def gather_wait(i, slot):
            pltpu.make_async_copy(table.at[islab.at[i]], bufs.at[slot],
                                  sem_g.at[slot]).wait()

        def write_issue(i, slot):
            pltpu.async_copy(bufs.at[slot],
                             g_hbm.at[c, pl.ds(base0 + i * _CB, _CB)],
                             sem_w.at[slot])

        def write_wait(i, slot):
            pltpu.make_async_copy(bufs.at[slot],
                                  g_hbm.at[c, pl.ds(base0 + i * _CB, _CB)],
                                  sem_w.at[slot]).wait()

        gather_issue(0, 0)
        gather_issue(1, 1)

        def body(grp, carry):
            for b in range(_NSLOT):
                i = grp * _NSLOT + b
                gather_wait(i, b)
                write_issue(i, b)
                nb = (b + 2) % _NSLOT

                @pl.when(i >= _NSLOT - 2)
                def _():
                    write_wait(i - (_NSLOT - 2), nb)

                @pl.when(i + 2 < nch)
                def _():
                    gather_issue(i + 2, nb)
            return carry

        lax.fori_loop(0, nch // _NSLOT, body, 0)
        for k in range(_NSLOT - 2):
            write_wait(nch - 1 - k, (nch - 1 - k) % _NSLOT)

    return gk(tab, idx4)


def _sc_scatter(e_new, idx3, n_nodes):
    e, h = e_new.shape
    nch = idx3.shape[1]
    epw = nch * _CBS
    zr = 32
    n_pad = ((n_nodes + zr * _NS - 1) // (zr * _NS)) * (zr * _NS)
    rpt = n_pad // _NS
    mesh = plsc.VectorSubcoreMesh(core_axis_name="c", subcore_axis_name="s")

    @functools.partial(
        pl.kernel,
        mesh=mesh,
        out_type=jax.ShapeDtypeStruct((_NC, n_pad, h), jnp.float32),
        scratch_types=[
            pltpu.VMEM((nch, _CBS), jnp.int32),
            pltpu.VMEM((2, _CBS, h), jnp.float32),
            pltpu.VMEM((zr, h), jnp.float32),
            pltpu.VMEM_SHARED((n_pad, h), jnp.float32),
            pltpu.SemaphoreType.DMA,
            pltpu.SemaphoreType.DMA((2,)),
            pltpu.SemaphoreType.DMA((2,)),
        ],
    )
    def sk(e_hbm, idx_hbm, out_hbm, islab, bufs, zbuf, agg,
           sem_i, sem_l, sem_s):
        c = lax.axis_index("c")
        s = lax.axis_index("s")
        wid = s * _NC + c
        cp_idx = pltpu.async_copy(idx_hbm.at[wid], islab, sem_i)

        for r in range(zr):
            for k in range(h // 16):
                zbuf[r, pl.ds(k * 16, 16)] = jnp.zeros((16,), jnp.float32)
        for j in range(rpt // zr):
            pltpu.sync_copy(zbuf, agg.at[pl.ds(s * rpt + j * zr, zr)])
        cp_idx.wait()
        plsc.subcore_barrier()

        base0 = wid * epw

        def load_issue(i, slot):
            pltpu.async_copy(e_hbm.at[pl.ds(base0 + i * _CBS, _CBS)],
                             bufs.at[slot], sem_l.at[slot])

        def load_wait(i, slot):
            pltpu.make_async_copy(e_hbm.at[pl.ds(base0 + i * _CBS, _CBS)],
                                  bufs.at[slot], sem_l.at[slot]).wait()

        def scat_issue(i, slot):
            pltpu.async_copy(bufs.at[slot], agg.at[islab.at[i]],
                             sem_s.at[slot], add=True)

        def scat_wait(i, slot):
            pltpu.make_async_copy(bufs.at[slot], agg.at[islab.at[i]],
                                  sem_s.at[slot]).wait()

        load_issue(0, 0)

        def body(grp, carry):
            for b in range(2):
                i = grp * 2 + b
                load_wait(i, b)
                scat_issue(i, b)
                nb = 1 - b

                @pl.when(i >= 1)
                def _():
                    scat_wait(i - 1, nb)

                @pl.when(i + 1 < nch)
                def _():
                    load_issue(i + 1, nb)
            return carry

        lax.fori_loop(0, nch // 2, body, 0)
        scat_wait(nch - 1, (nch - 1) % 2)
        plsc.subcore_barrier()
        pltpu.sync_copy(
            agg.at[pl.ds(s * rpt, rpt)],
            out_hbm.at[c, pl.ds(s * rpt, rpt)],
        )

    return sk(e_new, idx3)


def kernel(x, edge_attr, edge_index, params):
    n, h = x.shape
    e = edge_attr.shape[0]
    src = edge_index[0]
    dst = edge_index[1]
    bn = 2000
    be = 4000

    nch_g = e // (_NS * _CB)
    nch_s = e // (_NW * _CBS)
    idx4 = jnp.stack([src.reshape(_NS, nch_g, _CB),
                      dst.reshape(_NS, nch_g, _CB)])
    idx3 = dst.reshape(_NW, nch_s, _CBS)

    cx, ce = x, edge_attr
    for p in params:
        we1 = p["We1"]
        a, b, c = we1[0:h], we1[h:2 * h], we1[2 * h:3 * h]
        wn1 = p["Wn1"]
        d, f = wn1[0:h], wn1[h:2 * h]
        be1 = p["be1"].reshape(1, h)
        be2 = p["be2"].reshape(1, h)
        bn1 = p["bn1"].reshape(1, h)
        bn2 = p["bn2"].reshape(1, h)

        tab = _tc_proj(cx, a, b, bn)
        g = _sc_gather(tab, idx4)
        ce = _tc_edge(g, ce, c, p["We2"], be1, be2, be)
        part = _sc_scatter(ce, idx3, n)
        cx = _tc_node(cx, part, d, f, p["Wn2"], bn1, bn2, bn)

    return (cx, ce)


# speedup vs baseline: 1.0899x; 1.0051x over previous
import functools

import jax
import jax.numpy as jnp
from jax import lax
from jax.experimental import pallas as pl
from jax.experimental.pallas import tpu as pltpu
from jax.experimental.pallas import tpu_sc as plsc

_NC = 2
_NS = 16
_NW = _NC * _NS
_CB = 80


def _proj_body(x_ref, a_ref, b_ref, tab_ref):
    x = x_ref[...]
    tab_ref[0] = jnp.dot(x, a_ref[...], preferred_element_type=jnp.float32)
    tab_ref[1] = jnp.dot(x, b_ref[...], preferred_element_type=jnp.float32)


def _tc_proj(x, a, b, bn):
    n, h = x.shape
    grid = (n // bn,)
    row = lambda i: (i, 0)
    zero = lambda i: (0, 0)
    return pl.pallas_call(
        _proj_body,
        grid=grid,
        in_specs=[
            pl.BlockSpec((bn, h), row),
            pl.BlockSpec((h, h), zero),
            pl.BlockSpec((h, h), zero),
        ],
        out_specs=pl.BlockSpec((2, bn, h), lambda i: (0, i, 0)),
        out_shape=jax.ShapeDtypeStruct((2, n, h), jnp.float32),
    )(x, a, b)


def _edge_body(g_ref, e_ref, c_ref, w2_ref, b1_ref, b2_ref, out_ref):
    e = e_ref[...]
    pre = (
        g_ref[0]
        + g_ref[1]
        + jnp.dot(e, c_ref[...], preferred_element_type=jnp.float32)
        + b1_ref[...]
    )
    h = jnp.maximum(pre, 0.0)
    out_ref[...] = (
        jnp.dot(h, w2_ref[...], preferred_element_type=jnp.float32)
        + b2_ref[...]
        + e
    )


def _tc_edge(g, e, c, w2, b1, b2, be):
    m, h = e.shape
    grid = (m // be,)
    row = lambda i: (i, 0)
    zero = lambda i: (0, 0)
    return pl.pallas_call(
        _edge_body,
        grid=grid,
        in_specs=[
            pl.BlockSpec((2, be, h), lambda i: (0, i, 0)),
            pl.BlockSpec((be, h), row),
            pl.BlockSpec((h, h), zero),
            pl.BlockSpec((h, h), zero),
            pl.BlockSpec((1, h), zero),
            pl.BlockSpec((1, h), zero),
        ],
        out_specs=pl.BlockSpec((be, h), row),
        out_shape=jax.ShapeDtypeStruct((m, h), jnp.float32),
    )(g, e, c, w2, b1, b2)


def _node_body(x_ref, p_ref, d_ref, f_ref, w2_ref, b1_ref, b2_ref, out_ref):
    x = x_ref[...]
    agg = p_ref[0] + p_ref[1]
    pre = (
        jnp.dot(x, d_ref[...], preferred_element_type=jnp.float32)
        + jnp.dot(agg, f_ref[...], preferred_element_type=jnp.float32)
        + b1_ref[...]
    )
    h = jnp.maximum(pre, 0.0)
    out_ref[...] = (
        jnp.dot(h, w2_ref[...], preferred_element_type=jnp.float32)
        + b2_ref[...]
        + x
    )


def _tc_node(x, part, d, f, w2, b1, b2, bn):
    n, h = x.shape
    grid = (n // bn,)
    row = lambda i: (i, 0)
    zero = lambda i: (0, 0)
    return pl.pallas_call(
        _node_body,
        grid=grid,
        in_specs=[
            pl.BlockSpec((bn, h), row),
            pl.BlockSpec((2, bn, h), lambda i: (0, i, 0)),
            pl.BlockSpec((h, h), zero),
            pl.BlockSpec((h, h), zero),
            pl.BlockSpec((h, h), zero),
            pl.BlockSpec((1, h), zero),
            pl.BlockSpec((1, h), zero),
        ],
        out_specs=pl.BlockSpec((bn, h), row),
        out_shape=jax.ShapeDtypeStruct((n, h), jnp.float32),
    )(x, part, d, f, w2, b1, b2)


_NSLOT = 5
_CBS = 40


def _sc_gather(tab, idx4):
    n, h = tab.shape[1], tab.shape[2]
    nch = idx4.shape[2]
    ept = nch * _CB
    e = ept * _NS
    mesh = plsc.VectorSubcoreMesh(core_axis_name="c", subcore_axis_name="s")

    @functools.partial(
        pl.kernel,
        mesh=mesh,
        out_type=jax.ShapeDtypeStruct((2, e, h), jnp.float32),
        scratch_types=[
            pltpu.VMEM((nch, _CB), jnp.int32),
            pltpu.VMEM((_NSLOT, _CB, h), jnp.float32),
            pltpu.SemaphoreType.DMA,
            pltpu.SemaphoreType.DMA((_NSLOT,)),
            pltpu.SemaphoreType.DMA((_NSLOT,)),
        ],
    )
    def gk(tab_hbm, idx_hbm, g_hbm, islab, bufs, sem_i, sem_g, sem_w):
        c = lax.axis_index("c")
        s = lax.axis_index("s")
        table = tab_hbm.at[c]
        cp_idx = pltpu.async_copy(idx_hbm.at[c, s], islab, sem_i)
        cp_idx.wait()

        base0 = s * ept

        def gather_issue(i, slot):
            pltpu.async_copy(table.at[islab.at[i]], bufs.at[slot],
                             sem_g.at[slot])

        def gather_wait(i, slot):
            pltpu.make_async_copy(table.at[islab.at[i]], bufs.at[slot],
                                  sem_g.at[slot]).wait()

        def write_issue(i, slot):
            pltpu.async_copy(bufs.at[slot],
                             g_hbm.at[c, pl.ds(base0 + i * _CB, _CB)],
                             sem_w.at[slot])

        def write_wait(i, slot):
            pltpu.make_async_copy(bufs.at[slot],
                                  g_hbm.at[c, pl.ds(base0 + i * _CB, _CB)],
                                  sem_w.at[slot]).wait()

        gather_issue(0, 0)
        gather_issue(1, 1)

        def body(grp, carry):
            for b in range(_NSLOT):
                i = grp * _NSLOT + b
                gather_wait(i, b)
                write_issue(i, b)
                nb = (b + 2) % _NSLOT

                @pl.when(i >= _NSLOT - 2)
                def _():
                    write_wait(i - (_NSLOT - 2), nb)

                @pl.when(i + 2 < nch)
                def _():
                    gather_issue(i + 2, nb)
            return carry

        lax.fori_loop(0, nch // _NSLOT, body, 0)
        for k in range(_NSLOT - 2):
            write_wait(nch - 1 - k, (nch - 1 - k) % _NSLOT)

    return gk(tab, idx4)


def _sc_scatter(e_new, idx3, n_nodes):
    e, h = e_new.shape
    nch = idx3.shape[1]
    epw = nch * _CBS
    zr = 32
    n_pad = ((n_nodes + zr * _NS - 1) // (zr * _NS)) * (zr * _NS)
    rpt = n_pad // _NS
    mesh = plsc.VectorSubcoreMesh(core_axis_name="c", subcore_axis_name="s")

    @functools.partial(
        pl.kernel,
        mesh=mesh,
        out_type=jax.ShapeDtypeStruct((_NC, n_pad, h), jnp.float32),
        scratch_types=[
            pltpu.VMEM((nch, _CBS), jnp.int32),
            pltpu.VMEM((2, _CBS, h), jnp.float32),
            pltpu.VMEM((zr, h), jnp.float32),
            pltpu.VMEM_SHARED((n_pad, h), jnp.float32),
            pltpu.SemaphoreType.DMA,
            pltpu.SemaphoreType.DMA((2,)),
            pltpu.SemaphoreType.DMA((2,)),
        ],
    )
    def sk(e_hbm, idx_hbm, out_hbm, islab, bufs, zbuf, agg,
           sem_i, sem_l, sem_s):
        c = lax.axis_index("c")
        s = lax.axis_index("s")
        wid = s * _NC + c
        cp_idx = pltpu.async_copy(idx_hbm.at[wid], islab, sem_i)

        for r in range(zr):
            for k in range(h // 16):
                zbuf[r, pl.ds(k * 16, 16)] = jnp.zeros((16,), jnp.float32)
        for j in range(rpt // zr):
            pltpu.sync_copy(zbuf, agg.at[pl.ds(s * rpt + j * zr, zr)])
        cp_idx.wait()
        plsc.subcore_barrier()

        base0 = wid * epw

        def load_issue(i, slot):
            pltpu.async_copy(e_hbm.at[pl.ds(base0 + i * _CBS, _CBS)],
                             bufs.at[slot], sem_l.at[slot])

        def load_wait(i, slot):
            pltpu.make_async_copy(e_hbm.at[pl.ds(base0 + i * _CBS, _CBS)],
                                  bufs.at[slot], sem_l.at[slot]).wait()

        def scat_issue(i, slot):
            pltpu.async_copy(bufs.at[slot], agg.at[islab.at[i]],
                             sem_s.at[slot], add=True)

        def scat_wait(i, slot):
            pltpu.make_async_copy(bufs.at[slot], agg.at[islab.at[i]],
                                  sem_s.at[slot]).wait()

        load_issue(0, 0)

        def body(grp, carry):
            for b in range(2):
                i = grp * 2 + b
                load_wait(i, b)
                scat_issue(i, b)
                nb = 1 - b

                @pl.when(i >= 1)
                def _():
                    scat_wait(i - 1, nb)

                @pl.when(i + 1 < nch)
                def _():
                    load_issue(i + 1, nb)
            return carry

        lax.fori_loop(0, nch // 2, body, 0)
        scat_wait(nch - 1, (nch - 1) % 2)
        plsc.subcore_barrier()
        pltpu.sync_copy(
            agg.at[pl.ds(s * rpt, rpt)],
            out_hbm.at[c, pl.ds(s * rpt, rpt)],
        )

    return sk(e_new, idx3)


def kernel(x, edge_attr, edge_index, params):
    n, h = x.shape
    e = edge_attr.shape[0]
    src = edge_index[0]
    dst = edge_index[1]
    bn = 2000
    be = 8000

    nch_g = e // (_NS * _CB)
    nch_s = e // (_NW * _CBS)
    idx4 = jnp.stack([src.reshape(_NS, nch_g, _CB),
                      dst.reshape(_NS, nch_g, _CB)])
    idx3 = dst.reshape(_NW, nch_s, _CBS)

    cx, ce = x, edge_attr
    for p in params:
        we1 = p["We1"]
        a, b, c = we1[0:h], we1[h:2 * h], we1[2 * h:3 * h]
        wn1 = p["Wn1"]
        d, f = wn1[0:h], wn1[h:2 * h]
        be1 = p["be1"].reshape(1, h)
        be2 = p["be2"].reshape(1, h)
        bn1 = p["bn1"].reshape(1, h)
        bn2 = p["bn2"].reshape(1, h)

        tab = _tc_proj(cx, a, b, bn)
        g = _sc_gather(tab, idx4)
        ce = _tc_edge(g, ce, c, p["We2"], be1, be2, be)
        part = _sc_scatter(ce, idx3, n)
        cx = _tc_node(cx, part, d, f, p["Wn2"], bn1, bn2, bn)

    return (cx, ce)


# speedup vs baseline: 1.0911x; 1.0011x over previous
import functools

import jax
import jax.numpy as jnp
from jax import lax
from jax.experimental import pallas as pl
from jax.experimental.pallas import tpu as pltpu
from jax.experimental.pallas import tpu_sc as plsc

_NC = 2
_NS = 16
_NW = _NC * _NS
_CB = 80


def _proj_body(x_ref, a_ref, b_ref, tab_ref):
    x = x_ref[...]
    tab_ref[0] = jnp.dot(x, a_ref[...], preferred_element_type=jnp.float32)
    tab_ref[1] = jnp.dot(x, b_ref[...], preferred_element_type=jnp.float32)


def _tc_proj(x, a, b, bn):
    n, h = x.shape
    grid = (n // bn,)
    row = lambda i: (i, 0)
    zero = lambda i: (0, 0)
    return pl.pallas_call(
        _proj_body,
        grid=grid,
        in_specs=[
            pl.BlockSpec((bn, h), row),
            pl.BlockSpec((h, h), zero),
            pl.BlockSpec((h, h), zero),
        ],
        out_specs=pl.BlockSpec((2, bn, h), lambda i: (0, i, 0)),
        out_shape=jax.ShapeDtypeStruct((2, n, h), jnp.float32),
    )(x, a, b)


def _edge_body(g_ref, e_ref, c_ref, w2_ref, b1_ref, b2_ref, out_ref):
    e = e_ref[...]
    pre = (
        g_ref[0]
        + g_ref[1]
        + jnp.dot(e, c_ref[...], preferred_element_type=jnp.float32)
        + b1_ref[...]
    )
    h = jnp.maximum(pre, 0.0)
    out_ref[...] = (
        jnp.dot(h, w2_ref[...], preferred_element_type=jnp.float32)
        + b2_ref[...]
        + e
    )


def _tc_edge(g, e, c, w2, b1, b2, be):
    m, h = e.shape
    grid = (m // be,)
    row = lambda i: (i, 0)
    zero = lambda i: (0, 0)
    return pl.pallas_call(
        _edge_body,
        grid=grid,
        in_specs=[
            pl.BlockSpec((2, be, h), lambda i: (0, i, 0)),
            pl.BlockSpec((be, h), row),
            pl.BlockSpec((h, h), zero),
            pl.BlockSpec((h, h), zero),
            pl.BlockSpec((1, h), zero),
            pl.BlockSpec((1, h), zero),
        ],
        out_specs=pl.BlockSpec((be, h), row),
        out_shape=jax.ShapeDtypeStruct((m, h), jnp.float32),
    )(g, e, c, w2, b1, b2)


def _node_body(x_ref, p_ref, d_ref, f_ref, w2_ref, b1_ref, b2_ref, out_ref):
    x = x_ref[...]
    agg = p_ref[0] + p_ref[1]
    pre = (
        jnp.dot(x, d_ref[...], preferred_element_type=jnp.float32)
        + jnp.dot(agg, f_ref[...], preferred_element_type=jnp.float32)
        + b1_ref[...]
    )
    h = jnp.maximum(pre, 0.0)
    out_ref[...] = (
        jnp.dot(h, w2_ref[...], preferred_element_type=jnp.float32)
        + b2_ref[...]
        + x
    )


def _node_proj_body(x_ref, p_ref, d_ref, f_ref, w2_ref, b1_ref, b2_ref,
                    a_ref, b_ref, out_ref, tab_ref):
    x = x_ref[...]
    agg = p_ref[0] + p_ref[1]
    pre = (
        jnp.dot(x, d_ref[...], preferred_element_type=jnp.float32)
        + jnp.dot(agg, f_ref[...], preferred_element_type=jnp.float32)
        + b1_ref[...]
    )
    h = jnp.maximum(pre, 0.0)
    xn = (
        jnp.dot(h, w2_ref[...], preferred_element_type=jnp.float32)
        + b2_ref[...]
        + x
    )
    out_ref[...] = xn
    tab_ref[0] = jnp.dot(xn, a_ref[...], preferred_element_type=jnp.float32)
    tab_ref[1] = jnp.dot(xn, b_ref[...], preferred_element_type=jnp.float32)


def _tc_node(x, part, d, f, w2, b1, b2, bn, ab_next=None):
    n, h = x.shape
    grid = (n // bn,)
    row = lambda i: (i, 0)
    zero = lambda i: (0, 0)
    in_specs = [
        pl.BlockSpec((bn, h), row),
        pl.BlockSpec((2, bn, h), lambda i: (0, i, 0)),
        pl.BlockSpec((h, h), zero),
        pl.BlockSpec((h, h), zero),
        pl.BlockSpec((h, h), zero),
        pl.BlockSpec((1, h), zero),
        pl.BlockSpec((1, h), zero),
    ]
    if ab_next is None:
        return pl.pallas_call(
            _node_body,
            grid=grid,
            in_specs=in_specs,
            out_specs=pl.BlockSpec((bn, h), row),
            out_shape=jax.ShapeDtypeStruct((n, h), jnp.float32),
        )(x, part, d, f, w2, b1, b2)
    a_next, b_next = ab_next
    return pl.pallas_call(
        _node_proj_body,
        grid=grid,
        in_specs=in_specs + [
            pl.BlockSpec((h, h), zero),
            pl.BlockSpec((h, h), zero),
        ],
        out_specs=[
            pl.BlockSpec((bn, h), row),
            pl.BlockSpec((2, bn, h), lambda i: (0, i, 0)),
        ],
        out_shape=[
            jax.ShapeDtypeStruct((n, h), jnp.float32),
            jax.ShapeDtypeStruct((2, n, h), jnp.float32),
        ],
    )(x, part, d, f, w2, b1, b2, a_next, b_next)


_NSLOT = 5
_CBS = 40


def _sc_gather(tab, idx4):
    n, h = tab.shape[1], tab.shape[2]
    nch = idx4.shape[2]
    ept = nch * _CB
    e = ept * _NS
    mesh = plsc.VectorSubcoreMesh(core_axis_name="c", subcore_axis_name="s")

    @functools.partial(
        pl.kernel,
        mesh=mesh,
        out_type=jax.ShapeDtypeStruct((2, e, h), jnp.float32),
        scratch_types=[
            pltpu.VMEM((nch, _CB), jnp.int32),
            pltpu.VMEM((_NSLOT, _CB, h), jnp.float32),
            pltpu.SemaphoreType.DMA,
            pltpu.SemaphoreType.DMA((_NSLOT,)),
            pltpu.SemaphoreType.DMA((_NSLOT,)),
        ],
    )
    def gk(tab_hbm, idx_hbm, g_hbm, islab, bufs, sem_i, sem_g, sem_w):
        c = lax.axis_index("c")
        s = lax.axis_index("s")
        table = tab_hbm.at[c]
        cp_idx = pltpu.async_copy(idx_hbm.at[c, s], islab, sem_i)
        cp_idx.wait()

        base0 = s * ept

        def gather_issue(i, slot):
            pltpu.async_copy(table.at[islab.at[i]], bufs.at[slot],
                             sem_g.at[slot])

        def gather_wait(i, slot):
            pltpu.make_async_copy(table.at[islab.at[i]], bufs.at[slot],
                                  sem_g.at[slot]).wait()

        def write_issue(i, slot):
            pltpu.async_copy(bufs.at[slot],
                             g_hbm.at[c, pl.ds(base0 + i * _CB, _CB)],
                             sem_w.at[slot])

        def write_wait(i, slot):
            pltpu.make_async_copy(bufs.at[slot],
                                  g_hbm.at[c, pl.ds(base0 + i * _CB, _CB)],
                                  sem_w.at[slot]).wait()

        gather_issue(0, 0)
        gather_issue(1, 1)

        def body(grp, carry):
            for b in range(_NSLOT):
                i = grp * _NSLOT + b
                gather_wait(i, b)
                write_issue(i, b)
                nb = (b + 2) % _NSLOT

                @pl.when(i >= _NSLOT - 2)
                def _():
                    write_wait(i - (_NSLOT - 2), nb)

                @pl.when(i + 2 < nch)
                def _():
                    gather_issue(i + 2, nb)
            return carry

        lax.fori_loop(0, nch // _NSLOT, body, 0)
        for k in range(_NSLOT - 2):
            write_wait(nch - 1 - k, (nch - 1 - k) % _NSLOT)

    return gk(tab, idx4)


def _sc_scatter(e_new, idx3, n_nodes):
    e, h = e_new.shape
    nch = idx3.shape[1]
    epw = nch * _CBS
    zr = 32
    n_pad = ((n_nodes + zr * _NS - 1) // (zr * _NS)) * (zr * _NS)
    rpt = n_pad // _NS
    mesh = plsc.VectorSubcoreMesh(core_axis_name="c", subcore_axis_name="s")

    @functools.partial(
        pl.kernel,
        mesh=mesh,
        out_type=jax.ShapeDtypeStruct((_NC, n_pad, h), jnp.float32),
        scratch_types=[
            pltpu.VMEM((nch, _CBS), jnp.int32),
            pltpu.VMEM((2, _CBS, h), jnp.float32),
            pltpu.VMEM((zr, h), jnp.float32),
            pltpu.VMEM_SHARED((n_pad, h), jnp.float32),
            pltpu.SemaphoreType.DMA,
            pltpu.SemaphoreType.DMA((2,)),
            pltpu.SemaphoreType.DMA((2,)),
        ],
    )
    def sk(e_hbm, idx_hbm, out_hbm, islab, bufs, zbuf, agg,
           sem_i, sem_l, sem_s):
        c = lax.axis_index("c")
        s = lax.axis_index("s")
        wid = s * _NC + c
        cp_idx = pltpu.async_copy(idx_hbm.at[wid], islab, sem_i)

        for r in range(zr):
            for k in range(h // 16):
                zbuf[r, pl.ds(k * 16, 16)] = jnp.zeros((16,), jnp.float32)
        for j in range(rpt // zr):
            pltpu.sync_copy(zbuf, agg.at[pl.ds(s * rpt + j * zr, zr)])
        cp_idx.wait()
        plsc.subcore_barrier()

        base0 = wid * epw

        def load_issue(i, slot):
            pltpu.async_copy(e_hbm.at[pl.ds(base0 + i * _CBS, _CBS)],
                             bufs.at[slot], sem_l.at[slot])

        def load_wait(i, slot):
            pltpu.make_async_copy(e_hbm.at[pl.ds(base0 + i * _CBS, _CBS)],
                                  bufs.at[slot], sem_l.at[slot]).wait()

        def scat_issue(i, slot):
            pltpu.async_copy(bufs.at[slot], agg.at[islab.at[i]],
                             sem_s.at[slot], add=True)

        def scat_wait(i, slot):
            pltpu.make_async_copy(bufs.at[slot], agg.at[islab.at[i]],
                                  sem_s.at[slot]).wait()

        load_issue(0, 0)

        def body(grp, carry):
            for b in range(2):
                i = grp * 2 + b
                load_wait(i, b)
                scat_issue(i, b)
                nb = 1 - b

                @pl.when(i >= 1)
                def _():
                    scat_wait(i - 1, nb)

                @pl.when(i + 1 < nch)
                def _():
                    load_issue(i + 1, nb)
            return carry

        lax.fori_loop(0, nch // 2, body, 0)
        scat_wait(nch - 1, (nch - 1) % 2)
        plsc.subcore_barrier()
        pltpu.sync_copy(
            agg.at[pl.ds(s * rpt, rpt)],
            out_hbm.at[c, pl.ds(s * rpt, rpt)],
        )

    return sk(e_new, idx3)


def kernel(x, edge_attr, edge_index, params):
    n, h = x.shape
    e = edge_attr.shape[0]
    src = edge_index[0]
    dst = edge_index[1]
    bn = 2000
    be = 8000

    nch_g = e // (_NS * _CB)
    nch_s = e // (_NW * _CBS)
    idx4 = jnp.stack([src.reshape(_NS, nch_g, _CB),
                      dst.reshape(_NS, nch_g, _CB)])
    idx3 = dst.reshape(_NW, nch_s, _CBS)

    cx, ce = x, edge_attr
    nb = len(params)
    tab = None
    for bi, p in enumerate(params):
        we1 = p["We1"]
        a, b, c = we1[0:h], we1[h:2 * h], we1[2 * h:3 * h]
        wn1 = p["Wn1"]
        d, f = wn1[0:h], wn1[h:2 * h]
        be1 = p["be1"].reshape(1, h)
        be2 = p["be2"].reshape(1, h)
        bn1 = p["bn1"].reshape(1, h)
        bn2 = p["bn2"].reshape(1, h)

        if tab is None:
            tab = _tc_proj(cx, a, b, bn)
        g = _sc_gather(tab, idx4)
        ce = _tc_edge(g, ce, c, p["We2"], be1, be2, be)
        part = _sc_scatter(ce, idx3, n)
        if bi + 1 < nb:
            we1n = params[bi + 1]["We1"]
            cx, tab = _tc_node(cx, part, d, f, p["Wn2"], bn1, bn2, bn,
                               ab_next=(we1n[0:h], we1n[h:2 * h]))
        else:
            cx = _tc_node(cx, part, d, f, p["Wn2"], bn1, bn2, bn)

    return (cx, ce)
